# baseline (speedup 1.0000x reference)
import jax
import jax.numpy as jnp
import numpy as np
from jax import lax
from jax.experimental import pallas as pl
from jax.experimental.pallas import tpu as pltpu
from jax.experimental.pallas import tpu_sc as plsc

_N = 100000
_NUM_CLASSES = 100
_D = 256
_POS3D = 84
_TEMP = 10000.0

_NCORES = 2
_NSUB = 16
_NW = _NCORES * _NSUB
_PW = 3200
_NPAD = _NW * _PW
_CH = 160
_NCHUNK = _PW // _CH
_TAB = _NUM_CLASSES * _D
_CHD = _CH * _D

_NPAIR = 126

_SIN_C = (6.253743606182701, -39.1618487014363, 57.02564859022385)
_COS_C = (0.9985667768466333, -19.55273513599132, 61.10729715875407,
          -59.580280760342845)


def _psegs_for(g):
    lo, hi = 16 * g, 16 * g + 16
    out = []
    for s, (a, b) in enumerate(((0, 42), (42, 84), (84, 126))):
        if lo < b and hi > a:
            out.append(s)
    return tuple(out)


def _sc_body(xs, ys, zs, lab, a0, a1, a2, bb, table, out,
             xs_v, ys_v, zs_v, lab_v, a0_v, a1_v, a2_v, b_v, tab_v,
             buf0, buf1, wsem0, wsem1):
    wid = lax.axis_index("s") * _NCORES + lax.axis_index("c")
    base = wid * _PW

    pltpu.sync_copy(xs.at[pl.ds(base, _PW)], xs_v)
    pltpu.sync_copy(ys.at[pl.ds(base, _PW)], ys_v)
    pltpu.sync_copy(zs.at[pl.ds(base, _PW)], zs_v)
    pltpu.sync_copy(lab.at[pl.ds(base, _PW)], lab_v)
    pltpu.sync_copy(a0, a0_v)
    pltpu.sync_copy(a1, a1_v)
    pltpu.sync_copy(a2, a2_v)
    pltpu.sync_copy(bb, b_v)
    pltpu.sync_copy(table, tab_v)

    coord_vs = (ys_v, xs_v, zs_v)
    a_vs = (a0_v, a1_v, a2_v)

    magic = jnp.float32(12582912.0)
    iota16 = lax.iota(jnp.int32, 16)
    iota2 = iota16 * 2
    m14 = iota16 < 14
    m4 = iota16 < 4

    def stage(ci, buf, wsem):
        row0 = base + ci * _CH
        rowp = row0 - 2 * _CH

        @pl.when(jnp.logical_and(ci >= 2, rowp + _CH <= _N))
        def _wait_full():
            pltpu.make_async_copy(buf, out.at[pl.ds(rowp * _D, _CHD)],
                                  wsem).wait()

        for glo in (0, 4):
            b_regs = {g: b_v[pl.ds(16 * g, 16)] for g in range(glo, glo + 4)}
            a_regs = {g: [(s, a_vs[s][pl.ds(16 * g, 16)])
                          for s in _psegs_for(g)]
                      for g in range(glo, glo + 4)}
            segs_used = sorted({s for g in range(glo, glo + 4)
                                for s in _psegs_for(g)})

            def one_point(p, gl=glo, br=b_regs, ar=a_regs, su=segs_used):
                pp = ci * _CH + p
                pidx = jnp.full((16,), pp, jnp.int32)
                cb = {s: plsc.load_gather(coord_vs[s], [pidx]) for s in su}
                labv = plsc.load_gather(lab_v, [pidx])
                labbase = lax.shift_left(labv, 8)
                idx_e = labbase + iota2
                idx_o = idx_e + 1
                psplat = jnp.full((16,), p * _D, jnp.int32)
                sidx_e = psplat + iota2
                sidx_o = sidx_e + 1
                for g in range(gl, gl + 4):
                    t = br[g]
                    for (s, av) in ar[g]:
                        t = cb[s] * av + t
                    r = t - ((t + magic) - magic)
                    u = r * r
                    sin_v = r * ((jnp.float32(_SIN_C[2]) * u
                                  + jnp.float32(_SIN_C[1])) * u
                                 + jnp.float32(_SIN_C[0]))
                    cos_v = (((jnp.float32(_COS_C[3]) * u
                               + jnp.float32(_COS_C[2])) * u
                              + jnp.float32(_COS_C[1])) * u
                             + jnp.float32(_COS_C[0]))
                    msk = m14 if g == 7 else None
                    tref = tab_v.at[pl.ds(32 * g, _TAB - 32 * g)]
                    bref = buf.at[pl.ds(32 * g, _CHD - 32 * g)]
                    emb_e = plsc.load_gather(tref, [idx_e], mask=msk)
                    emb_o = plsc.load_gather(tref, [idx_o], mask=msk)
                    plsc.store_scatter(bref, [sidx_e], emb_e + sin_v,
                                       mask=msk)
                    plsc.store_scatter(bref, [sidx_o], emb_o + cos_v,
                                       mask=msk)
                if gl != 0:
                    idx_p = labbase + (iota16 + 4)
                    emb_p = plsc.load_gather(
                        tab_v.at[pl.ds(248, _TAB - 248)], [idx_p], mask=m4)
                    plsc.store_scatter(
                        buf.at[pl.ds(248, _CHD - 248)],
                        [psplat + (iota16 + 4)], emb_p, mask=m4)

            @plsc.parallel_loop(0, _CH, unroll=2)
            def _pts(p, op=one_point):
                op(p)

        @pl.when(row0 + _CH <= _N)
        def _full_write():
            pltpu.async_copy(buf, out.at[pl.ds(row0 * _D, _CHD)], wsem)

    def pair_body(it, carry):
        stage(it * 2, buf0, wsem0)
        stage(it * 2 + 1, buf1, wsem1)
        return carry

    lax.fori_loop(0, _NCHUNK // 2, pair_body, 0)

    for (cl, bufb, wsemb) in ((_NCHUNK - 2, buf0, wsem0),
                              (_NCHUNK - 1, buf1, wsem1)):
        rowl = base + cl * _CH

        @pl.when(rowl + _CH <= _N)
        def _drain_full(bufb=bufb, wsemb=wsemb, rowl=rowl):
            pltpu.make_async_copy(bufb, out.at[pl.ds(rowl * _D, _CHD)],
                                  wsemb).wait()


_sc_call = pl.kernel(
    _sc_body,
    out_type=jax.ShapeDtypeStruct((_N * _D,), jnp.float32),
    mesh=plsc.VectorSubcoreMesh(core_axis_name="c", subcore_axis_name="s"),
    scratch_types=[
        pltpu.VMEM((_PW,), jnp.float32),
        pltpu.VMEM((_PW,), jnp.float32),
        pltpu.VMEM((_PW,), jnp.float32),
        pltpu.VMEM((_PW,), jnp.int32),
        pltpu.VMEM((128,), jnp.float32),
        pltpu.VMEM((128,), jnp.float32),
        pltpu.VMEM((128,), jnp.float32),
        pltpu.VMEM((128,), jnp.float32),
        pltpu.VMEM((_TAB,), jnp.float32),
        pltpu.VMEM((_CHD,), jnp.float32),
        pltpu.VMEM((_CHD,), jnp.float32),
        pltpu.SemaphoreType.DMA,
        pltpu.SemaphoreType.DMA,
    ],
    compiler_params=pltpu.CompilerParams(needs_layout_passes=False),
)


def kernel(point_coord, labels, pc_range, label_embed_weight):
    pc = point_coord[0]
    padn = _NPAD - _N
    xs = jnp.pad(pc[:, 0], (0, padn))
    ys = jnp.pad(pc[:, 1], (0, padn))
    zs = jnp.pad(pc[:, 2], (0, padn))
    lab = jnp.pad(labels, (0, padn))

    off = pc_range[0:3]
    den = jnp.stack([pc_range[3] - pc_range[0],
                     pc_range[4] - pc_range[1],
                     pc_range[5] - pc_range[2]])

    i = np.arange(_POS3D // 2, dtype=np.float64)
    invf = (_TEMP ** (-(2.0 * i / _POS3D))).astype(np.float32)
    invf_j = jnp.asarray(invf)

    seg_a, seg_b = [], []
    for c in (1, 0, 2):
        a = invf_j / den[c]
        seg_a.append(a)
        seg_b.append(-off[c] * a)
    z42 = jnp.zeros((42,), jnp.float32)
    z2 = jnp.zeros((2,), jnp.float32)
    a0 = jnp.concatenate([seg_a[0], z42, z42, z2])
    a1 = jnp.concatenate([z42, seg_a[1], z42, z2])
    a2 = jnp.concatenate([z42, z42, seg_a[2], z2])
    bb = jnp.concatenate([seg_b[0], seg_b[1], seg_b[2], z2])

    flat = _sc_call(xs, ys, zs, lab, a0, a1, a2, bb,
                    label_embed_weight.reshape(-1))
    return flat.reshape(_N, _D)


# speedup vs baseline: 1.4932x; 1.4932x over previous
import jax
import jax.numpy as jnp
import numpy as np
from jax import lax
from jax.experimental import pallas as pl
from jax.experimental.pallas import tpu as pltpu
from jax.experimental.pallas import tpu_sc as plsc

_N = 100000
_NUM_CLASSES = 100
_D = 256
_POS3D = 84
_TEMP = 10000.0

_NCORES = 2
_NSUB = 16
_NW = _NCORES * _NSUB
_PW = 3200
_NPAD = _NW * _PW
_CH = 160
_NCHUNK = _PW // _CH

_SIN_COEFFS = (6.253743606182701, -39.1618487014363, 57.02564859022385)


def _segs_for(j):
    lo, hi = 16 * j, 16 * j + 16
    out = []
    for s, (a, b) in enumerate(((0, _POS3D), (_POS3D, 2 * _POS3D),
                                (2 * _POS3D, 3 * _POS3D))):
        if lo < b and hi > a:
            out.append(s)
    return tuple(out)


def _sc_body(xs, ys, zs, lab, a0, a1, a2, bb, table, out,
             xs_v, ys_v, zs_v, lab_v, a0_v, a1_v, a2_v, b_v, tab_v,
             buf0, buf1, wsem0, wsem1):
    wid = lax.axis_index("s") * _NCORES + lax.axis_index("c")
    base = wid * _PW

    pltpu.sync_copy(xs.at[pl.ds(base, _PW)], xs_v)
    pltpu.sync_copy(ys.at[pl.ds(base, _PW)], ys_v)
    pltpu.sync_copy(zs.at[pl.ds(base, _PW)], zs_v)
    pltpu.sync_copy(lab.at[pl.ds(base, _PW)], lab_v)
    pltpu.sync_copy(a0, a0_v)
    pltpu.sync_copy(a1, a1_v)
    pltpu.sync_copy(a2, a2_v)
    pltpu.sync_copy(bb, b_v)
    pltpu.sync_copy(table, tab_v)

    coord_vs = (ys_v, xs_v, zs_v)
    a_vs = (a0_v, a1_v, a2_v)

    magic = jnp.float32(12582912.0)
    iota16 = lax.iota(jnp.int32, 16)

    def stage(ci, buf, wsem):
        row0 = base + ci * _CH
        rowp = row0 - 2 * _CH

        @pl.when(jnp.logical_and(ci >= 2, rowp + _CH <= _N))
        def _wait_full():
            pltpu.make_async_copy(buf, out.at[pl.ds(rowp, _CH)], wsem).wait()

        for jlo in (0, 8):
            b_regs = {j: b_v[pl.ds(16 * j, 16)] for j in range(jlo, jlo + 8)}
            a_regs = {j: [(s, a_vs[s][pl.ds(16 * j, 16)]) for s in _segs_for(j)]
                      for j in range(jlo, jlo + 8)}
            segs_used = sorted({s for j in range(jlo, jlo + 8)
                                for s in _segs_for(j)})

            def one_point(p, jl=jlo, br=b_regs, ar=a_regs, su=segs_used):
                pp = ci * _CH + p
                pidx = jnp.full((16,), pp, jnp.int32)
                cb = {s: plsc.load_gather(coord_vs[s], [pidx]) for s in su}
                labv = plsc.load_gather(lab_v, [pidx])
                idx = lax.shift_left(labv, 8) + iota16
                for j in range(jl, jl + 8):
                    t = br[j]
                    for (s, av) in ar[j]:
                        t = cb[s] * av + t
                    r = t - ((t + magic) - magic)
                    u = r * r
                    pacc = jnp.float32(_SIN_COEFFS[2])
                    for cc in _SIN_COEFFS[1::-1]:
                        pacc = pacc * u + jnp.float32(cc)
                    sin_v = r * pacc
                    emb = plsc.load_gather(
                        tab_v.at[pl.ds(16 * j, _NUM_CLASSES * _D - 16 * j)],
                        [idx])
                    buf[p, pl.ds(16 * j, 16)] = emb + sin_v

            @plsc.parallel_loop(0, _CH, unroll=2)
            def _pts(p, op=one_point):
                op(p)

        @pl.when(row0 + _CH <= _N)
        def _full_write():
            pltpu.async_copy(buf, out.at[pl.ds(row0, _CH)], wsem)

    def pair_body(it, carry):
        stage(it * 2, buf0, wsem0)
        stage(it * 2 + 1, buf1, wsem1)
        return carry

    lax.fori_loop(0, _NCHUNK // 2, pair_body, 0)

    for (cl, bufb, wsemb) in ((_NCHUNK - 2, buf0, wsem0),
                              (_NCHUNK - 1, buf1, wsem1)):
        rowl = base + cl * _CH

        @pl.when(rowl + _CH <= _N)
        def _drain_full(bufb=bufb, wsemb=wsemb, rowl=rowl):
            pltpu.make_async_copy(bufb, out.at[pl.ds(rowl, _CH)], wsemb).wait()


_sc_call = pl.kernel(
    _sc_body,
    out_type=jax.ShapeDtypeStruct((_N, _D), jnp.float32),
    mesh=plsc.VectorSubcoreMesh(core_axis_name="c", subcore_axis_name="s"),
    scratch_types=[
        pltpu.VMEM((_PW,), jnp.float32),
        pltpu.VMEM((_PW,), jnp.float32),
        pltpu.VMEM((_PW,), jnp.float32),
        pltpu.VMEM((_PW,), jnp.int32),
        pltpu.VMEM((_D,), jnp.float32),
        pltpu.VMEM((_D,), jnp.float32),
        pltpu.VMEM((_D,), jnp.float32),
        pltpu.VMEM((_D,), jnp.float32),
        pltpu.VMEM((_NUM_CLASSES * _D,), jnp.float32),
        pltpu.VMEM((_CH, _D), jnp.float32),
        pltpu.VMEM((_CH, _D), jnp.float32),
        pltpu.SemaphoreType.DMA,
        pltpu.SemaphoreType.DMA,
    ],
    compiler_params=pltpu.CompilerParams(needs_layout_passes=False),
)


def kernel(point_coord, labels, pc_range, label_embed_weight):
    pc = point_coord[0]
    padn = _NPAD - _N
    xs = jnp.pad(pc[:, 0], (0, padn))
    ys = jnp.pad(pc[:, 1], (0, padn))
    zs = jnp.pad(pc[:, 2], (0, padn))
    lab = jnp.pad(labels, (0, padn))

    off = pc_range[0:3]
    den = jnp.stack([pc_range[3] - pc_range[0],
                     pc_range[4] - pc_range[1],
                     pc_range[5] - pc_range[2]])

    k = np.arange(_POS3D, dtype=np.float64)
    invf = (_TEMP ** (-(2.0 * np.floor(k / 2.0) / _POS3D))).astype(np.float32)
    phase = np.where(np.arange(_POS3D) % 2 == 1, 0.25, 0.0).astype(np.float32)
    invf_j = jnp.asarray(invf)
    phase_j = jnp.asarray(phase)

    seg_a, seg_b = [], []
    for c in (1, 0, 2):
        a = invf_j / den[c]
        seg_a.append(a)
        seg_b.append(phase_j - off[c] * a)
    z84 = jnp.zeros((_POS3D,), jnp.float32)
    z4 = jnp.zeros((_D - 3 * _POS3D,), jnp.float32)
    a0 = jnp.concatenate([seg_a[0], z84, z84, z4])
    a1 = jnp.concatenate([z84, seg_a[1], z84, z4])
    a2 = jnp.concatenate([z84, z84, seg_a[2], z4])
    bb = jnp.concatenate([seg_b[0], seg_b[1], seg_b[2], z4])

    return _sc_call(xs, ys, zs, lab, a0, a1, a2, bb,
                    label_embed_weight.reshape(-1))
